# baseline (speedup 1.0000x reference)
import functools

import jax
import jax.numpy as jnp
from jax import lax
from jax.experimental import pallas as pl
from jax.experimental.pallas import tpu as pltpu
from jax.experimental.pallas import tpu_sc as plsc

VPAD = 16
NC = 2
NS = 16
NW = NC * NS
CHUNK = 128


def _table_body(emb_ref, w2_ref, b2_ref, w31_ref, b31_ref, w32_ref, b32_ref,
                out_ref):
    e = emb_ref[...]
    e = jnp.concatenate(
        [e, jnp.zeros((VPAD - e.shape[0], e.shape[1]), e.dtype)], axis=0)
    t = e * jax.nn.sigmoid(e)
    h = jnp.dot(t, w2_ref[...], preferred_element_type=jnp.float32) + b2_ref[...]
    h = h * jax.nn.sigmoid(h)
    lat = w31_ref.shape[1]
    out_ref[:, :lat] = (
        jnp.dot(h, w31_ref[...], preferred_element_type=jnp.float32) + b31_ref[...]
    )
    out_ref[:, lat:] = (
        jnp.dot(h, w32_ref[...], preferred_element_type=jnp.float32) + b32_ref[...]
    )


def _make_gather(n_chunks, out_dim):
    mesh = plsc.VectorSubcoreMesh(core_axis_name="c", subcore_axis_name="s")
    b_per_w = n_chunks * CHUNK

    @functools.partial(
        pl.kernel,
        mesh=mesh,
        out_type=jax.ShapeDtypeStruct((NW, n_chunks, CHUNK, out_dim),
                                      jnp.float32),
        scratch_types=[
            pltpu.VMEM((b_per_w,), jnp.int32),
            pltpu.VMEM((n_chunks, CHUNK, out_dim), jnp.float32),
            pltpu.VMEM((VPAD, out_dim), jnp.float32),
            pltpu.VMEM_SHARED((VPAD, out_dim), jnp.float32),
            pltpu.SemaphoreType.DMA,
            pltpu.SemaphoreType.DMA,
        ],
    )
    def gather(table_hbm, idx_hbm, out_hbm, idx_v, rows_v, table_v, table_sh,
               gsem, wsem):
        sid = lax.axis_index("s")
        wid = sid * NC + lax.axis_index("c")
        @pl.when(sid == 0)
        def _stage():
            pltpu.sync_copy(table_hbm, table_v)
            pltpu.sync_copy(table_v, table_sh)

        pltpu.sync_copy(idx_hbm.at[pl.ds(wid * b_per_w, b_per_w)], idx_v)
        plsc.subcore_barrier()
        gathers = [
            pltpu.async_copy(table_sh.at[idx_v.at[pl.ds(j * CHUNK, CHUNK)]],
                             rows_v.at[j], gsem)
            for j in range(n_chunks)
        ]
        writes = []
        for j in range(n_chunks):
            gathers[j].wait()
            writes.append(
                pltpu.async_copy(rows_v.at[j], out_hbm.at[wid].at[j], wsem))
        for w in writes:
            w.wait()

    return gather


def kernel(x, emb, W2, b2, W31, b31, W32, b32):
    batch = x.shape[0]
    lat = W31.shape[1]
    out_dim = 2 * lat
    n_chunks = batch // (NW * CHUNK)

    table = pl.pallas_call(
        _table_body,
        out_shape=jax.ShapeDtypeStruct((VPAD, out_dim), jnp.float32),
    )(emb, W2, b2, W31, b31, W32, b32)

    out = _make_gather(n_chunks, out_dim)(table, x)
    return out.reshape(batch, out_dim)
